# baseline (speedup 1.0000x reference)
import functools

import jax
import jax.numpy as jnp
from jax import lax
from jax.experimental import pallas as pl
from jax.experimental.pallas import tpu as pltpu
from jax.experimental.pallas import tpu_sc as plsc

_CHUNK = 2048
_UNROLL = 8


def kernel(inputs, class_bias, global_bias):
    B, F = inputs.shape
    _, V = class_bias.shape
    info = plsc.get_sparse_core_info()
    NC = info.num_cores
    n_chunks = B // _CHUNK
    mesh = plsc.VectorSubcoreMesh(core_axis_name="c", subcore_axis_name="s")

    @functools.partial(
        pl.kernel,
        out_type=jax.ShapeDtypeStruct((F, B), jnp.float32),
        mesh=mesh,
        compiler_params=pltpu.CompilerParams(needs_layout_passes=False),
        scratch_types=[
            pltpu.VMEM((V,), jnp.float32),
            pltpu.VMEM((B,), jnp.int32),
            pltpu.VMEM((2 * _CHUNK,), jnp.float32),
            pltpu.VMEM((16,), jnp.float32),
            pltpu.SemaphoreType.DMA,
            pltpu.SemaphoreType.DMA,
            pltpu.SemaphoreType.DMA,
            pltpu.SemaphoreType.DMA,
        ],
    )
    def k(idx_hbm, cb_hbm, gb_hbm, out_hbm, tab_v, idx_v, out_v, gbs_v,
          sem_t, sem_i, sem_o0, sem_o1):
        wid = lax.axis_index("s") * NC + lax.axis_index("c")

        @pl.when(wid < F)
        def _():
            cp_tab = pltpu.make_async_copy(cb_hbm.at[wid], tab_v, sem_t)
            cp_tab.start()
            cp_idx = pltpu.make_async_copy(idx_hbm.at[wid], idx_v, sem_i)
            cp_idx.start()
            pltpu.async_copy(
                gb_hbm.at[jnp.full((16,), wid, jnp.int32)], gbs_v,
                sem_o0).wait()
            gbv = gbs_v[...]
            cp_idx.wait()
            cp_tab.wait()

            out_sems = (sem_o0, sem_o1)
            out_cps = []
            for c in range(n_chunks):
                base = (c % 2) * _CHUNK
                if c >= 2:
                    out_cps[c - 2].wait()

                @plsc.parallel_loop(0, _CHUNK, step=16, unroll=_UNROLL)
                def body(s, c=c, base=base):
                    iv = idx_v[pl.ds(c * _CHUNK + s, 16)]
                    out_v[pl.ds(base + s, 16)] = (
                        plsc.load_gather(tab_v, [iv]) + gbv)
                cp = pltpu.make_async_copy(
                    out_v.at[pl.ds(base, _CHUNK)],
                    out_hbm.at[wid, pl.ds(c * _CHUNK, _CHUNK)],
                    out_sems[c % 2])
                cp.start()
                out_cps.append(cp)
            out_cps[n_chunks - 2].wait()
            out_cps[n_chunks - 1].wait()

    idx_t = inputs.astype(jnp.int32).T
    out_t = k(idx_t, class_bias, global_bias)
    return out_t.T


# speedup vs baseline: 1.0161x; 1.0161x over previous
import functools

import jax
import jax.numpy as jnp
from jax import lax
from jax.experimental import pallas as pl
from jax.experimental.pallas import tpu as pltpu
from jax.experimental.pallas import tpu_sc as plsc

_CHUNK = 4096
_UNROLL = 8


def kernel(inputs, class_bias, global_bias):
    B, F = inputs.shape
    _, V = class_bias.shape
    info = plsc.get_sparse_core_info()
    NC = info.num_cores
    n_chunks = B // _CHUNK
    mesh = plsc.VectorSubcoreMesh(core_axis_name="c", subcore_axis_name="s")

    @functools.partial(
        pl.kernel,
        out_type=jax.ShapeDtypeStruct((F, B), jnp.float32),
        mesh=mesh,
        compiler_params=pltpu.CompilerParams(needs_layout_passes=False),
        scratch_types=[
            pltpu.VMEM((V,), jnp.float32),
            pltpu.VMEM((B,), jnp.int32),
            pltpu.VMEM((2 * _CHUNK,), jnp.float32),
            pltpu.VMEM((16,), jnp.float32),
            pltpu.SemaphoreType.DMA,
            pltpu.SemaphoreType.DMA,
            pltpu.SemaphoreType.DMA,
            pltpu.SemaphoreType.DMA,
        ],
    )
    def k(idx_hbm, cb_hbm, gb_hbm, out_hbm, tab_v, idx_v, out_v, gbs_v,
          sem_t, sem_i, sem_o0, sem_o1):
        wid = lax.axis_index("s") * NC + lax.axis_index("c")

        @pl.when(wid < F)
        def _():
            cp_tab = pltpu.make_async_copy(cb_hbm.at[wid], tab_v, sem_t)
            cp_tab.start()
            cp_idx = pltpu.make_async_copy(idx_hbm.at[wid], idx_v, sem_i)
            cp_idx.start()
            pltpu.async_copy(
                gb_hbm.at[jnp.full((16,), wid, jnp.int32)], gbs_v,
                sem_o0).wait()
            gbv = gbs_v[...]
            cp_idx.wait()
            cp_tab.wait()

            out_sems = (sem_o0, sem_o1)
            out_cps = []
            for c in range(n_chunks):
                base = (c % 2) * _CHUNK
                if c >= 2:
                    out_cps[c - 2].wait()

                @plsc.parallel_loop(0, _CHUNK, step=16, unroll=_UNROLL)
                def body(s, c=c, base=base):
                    iv = idx_v[pl.ds(c * _CHUNK + s, 16)]
                    out_v[pl.ds(base + s, 16)] = (
                        plsc.load_gather(tab_v, [iv]) + gbv)
                cp = pltpu.make_async_copy(
                    out_v.at[pl.ds(base, _CHUNK)],
                    out_hbm.at[wid, pl.ds(c * _CHUNK, _CHUNK)],
                    out_sems[c % 2])
                cp.start()
                out_cps.append(cp)
            out_cps[n_chunks - 2].wait()
            out_cps[n_chunks - 1].wait()

    idx_t = inputs.astype(jnp.int32).T
    out_t = k(idx_t, class_bias, global_bias)
    return out_t.T
